# baseline (speedup 1.0000x reference)
import functools

import jax
import jax.numpy as jnp
from jax.experimental import pallas as pl
from jax.experimental.pallas import tpu as pltpu

H = 64
NG = 256
_PREC = jax.lax.Precision.HIGHEST


def _prep_body(xpg_ref, cg_ref, ex_ref, wie_ref, msg0_ref, b_ref):
    ew = jnp.dot(ex_ref[...], wie_ref[...], preferred_element_type=jnp.float32,
                 precision=_PREC)
    msg0_ref[...] = jnp.maximum(xpg_ref[...] + ew, 0.0)
    b_ref[...] = cg_ref[...] + ew


def _edge_prep(xpg, cg, ex8, wie8):
    E = xpg.shape[0]
    BLK = 8000
    grid = (E // BLK,)
    bs = lambda w: pl.BlockSpec((BLK, w), lambda i: (i, 0))
    return pl.pallas_call(
        _prep_body,
        grid=grid,
        in_specs=[bs(H), bs(H), bs(8), pl.BlockSpec((8, H), lambda i: (0, 0))],
        out_specs=[bs(H), bs(H)],
        out_shape=[jax.ShapeDtypeStruct((E, H), jnp.float32),
                   jax.ShapeDtypeStruct((E, H), jnp.float32)],
    )(xpg, cg, ex8, wie8)


def _iter_body(g_ref, m_ref, b_ref, w2_ref, o_ref):
    t = g_ref[...] - m_ref[...]
    tw = jnp.dot(t, w2_ref[...], preferred_element_type=jnp.float32,
                 precision=_PREC)
    accum = jnp.concatenate([tw[:, H:], tw[:, :H]], axis=1)
    o_ref[...] = jnp.maximum(b_ref[...] + accum, 0.0)


def _edge_iter(g, msg, b, w2):
    EP = g.shape[0]
    BLK = 5000
    grid = (EP // BLK,)
    bs = pl.BlockSpec((BLK, 2 * H), lambda i: (i, 0))
    return pl.pallas_call(
        _iter_body,
        grid=grid,
        in_specs=[bs, bs, bs, pl.BlockSpec((2 * H, 2 * H), lambda i: (0, 0))],
        out_specs=bs,
        out_shape=jax.ShapeDtypeStruct((EP, 2 * H), jnp.float32),
    )(g, msg, b, w2)


def _final_body(x64_ref, m_ref, gid_ref, wo_ref, bo_ref, sums_ref, cnts_ref):
    i = pl.program_id(0)

    @pl.when(i == 0)
    def _():
        sums_ref[...] = jnp.zeros_like(sums_ref)
        cnts_ref[...] = jnp.zeros_like(cnts_ref)

    xm = jnp.concatenate([x64_ref[...], m_ref[...]], axis=1)
    h = jnp.maximum(
        jnp.dot(xm, wo_ref[...], preferred_element_type=jnp.float32,
                precision=_PREC) + bo_ref[...], 0.0)
    oneh = (gid_ref[...] == jax.lax.broadcasted_iota(jnp.int32, (1, NG), 1)
            ).astype(jnp.float32)
    sums_ref[...] += jax.lax.dot_general(
        oneh, h, (((0,), (0,)), ((), ())), preferred_element_type=jnp.float32,
        precision=_PREC)
    cnts_ref[...] += jnp.sum(oneh, axis=0, keepdims=True)


def _final(x64, m, gid2d, wo128, bo2d):
    Nn = x64.shape[0]
    BLK = 2000
    grid = (Nn // BLK,)
    bs = lambda w: pl.BlockSpec((BLK, w), lambda i: (i, 0))
    return pl.pallas_call(
        _final_body,
        grid=grid,
        in_specs=[bs(H), bs(H), pl.BlockSpec((BLK, 1), lambda i: (i, 0)),
                  pl.BlockSpec((2 * H, H), lambda i: (0, 0)),
                  pl.BlockSpec((1, H), lambda i: (0, 0))],
        out_specs=[pl.BlockSpec((NG, H), lambda i: (0, 0)),
                   pl.BlockSpec((1, NG), lambda i: (0, 0))],
        out_shape=[jax.ShapeDtypeStruct((NG, H), jnp.float32),
                   jax.ShapeDtypeStruct((1, NG), jnp.float32)],
    )(x64, m, gid2d, wo128, bo2d)


def kernel(x, edge_index, edge_x, tree_mess, tree_mess_tgt_nodes, graph_ids,
           W_i, W_h, W_o, b_o):
    n = x.shape[0]
    E = edge_index.shape[1]
    dst = edge_index[1]
    depth_minus_1 = 3

    node_alpha = jnp.zeros((n, H), x.dtype).at[tree_mess_tgt_nodes].add(tree_mess)
    xp = x @ W_i[:, :x.shape[1]].T
    c = xp + node_alpha @ W_h.T

    ex8 = jnp.pad(edge_x, ((0, 0), (0, 8 - edge_x.shape[1])))
    wie8 = jnp.pad(W_i[:, x.shape[1]:], ((0, 0), (0, 8 - edge_x.shape[1]))).T
    xpg = xp[dst].reshape(E // 2, 2, H)[:, ::-1, :].reshape(E, H)
    cg = c[dst].reshape(E // 2, 2, H)[:, ::-1, :].reshape(E, H)
    msg, b = _edge_prep(xpg, cg, ex8, wie8)

    w2 = jnp.kron(jnp.eye(2, dtype=jnp.float32), W_h.T)
    msg_p = msg.reshape(E // 2, 2 * H)
    b_p = b.reshape(E // 2, 2 * H)
    for _ in range(depth_minus_1):
        ns = jax.ops.segment_sum(msg_p.reshape(E, H), dst, num_segments=n)
        g = ns[dst].reshape(E // 2, 2 * H)
        msg_p = _edge_iter(g, msg_p, b_p, w2)

    ns = jax.ops.segment_sum(msg_p.reshape(E, H), dst, num_segments=n)
    m = ns + node_alpha

    x64 = jnp.pad(x, ((0, 0), (0, H - x.shape[1])))
    wo128 = jnp.concatenate(
        [jnp.pad(W_o[:, :x.shape[1]], ((0, 0), (0, H - x.shape[1]))),
         W_o[:, x.shape[1]:]], axis=1).T
    sums, cnts = _final(x64, m, graph_ids.reshape(n, 1).astype(jnp.int32),
                        wo128, b_o.reshape(1, H))
    return sums / jnp.maximum(cnts.reshape(NG, 1), 1.0)


# speedup vs baseline: 2.0076x; 2.0076x over previous
import functools

import jax
import jax.numpy as jnp
from jax import lax
from jax.experimental import pallas as pl
from jax.experimental.pallas import tpu as pltpu
from jax.experimental.pallas import tpu_sc as plsc

H = 64
NG = 256
NC, NS = 2, 16
NW = NC * NS
_PREC = jax.lax.Precision.HIGHEST


def _sc_gather(table, idx2d, width, ch):
    nch = idx2d.shape[0]
    e_out = nch * ch
    mesh = plsc.VectorSubcoreMesh(core_axis_name="c", subcore_axis_name="s")

    def body(table_hbm, idx_hbm, out_hbm, ib0, ib1, rb0, rb1,
             isem0, isem1, gsem0, gsem1, osem0, osem1):
        wid = lax.axis_index("s") * NC + lax.axis_index("c")
        ibufs, rbufs = (ib0, ib1), (rb0, rb1)
        isems, gsems, osems = (isem0, isem1), (gsem0, gsem1), (osem0, osem1)

        for b in (0, 1):
            ch0 = wid + NW * b

            @pl.when(ch0 < nch)
            def _():
                pltpu.make_async_copy(idx_hbm.at[ch0], ibufs[b], isems[b]).start()

        @pl.loop(wid, nch, step=2 * NW)
        def _(g):
            for b in (0, 1):
                cno = g + NW * b

                @pl.when(cno < nch)
                def _():
                    @pl.when(cno >= wid + 2 * NW)
                    def _():
                        pltpu.make_async_copy(
                            rbufs[b], out_hbm.at[pl.ds((cno - 2 * NW) * ch, ch)],
                            osems[b]).wait()

                    pltpu.make_async_copy(idx_hbm.at[cno], ibufs[b],
                                          isems[b]).wait()
                    pltpu.make_async_copy(table_hbm.at[ibufs[b]], rbufs[b],
                                          gsems[b]).start()
                    pltpu.make_async_copy(table_hbm.at[ibufs[b]], rbufs[b],
                                          gsems[b]).wait()

                    @pl.when(cno + 2 * NW < nch)
                    def _():
                        pltpu.make_async_copy(idx_hbm.at[cno + 2 * NW],
                                              ibufs[b], isems[b]).start()

                    pltpu.make_async_copy(
                        rbufs[b], out_hbm.at[pl.ds(cno * ch, ch)],
                        osems[b]).start()

        for b in (0, 1):
            last = ((nch - 1 - wid - NW * b) // (2 * NW)) * 2 * NW + wid + NW * b

            @pl.when((last >= 0) & (last < nch))
            def _():
                pltpu.make_async_copy(
                    rbufs[b], out_hbm.at[pl.ds(last * ch, ch)], osems[b]).wait()

    f = pl.kernel(
        body, mesh=mesh,
        out_type=jax.ShapeDtypeStruct((e_out, width), jnp.float32),
        compiler_params=pltpu.CompilerParams(use_tc_tiling_on_sc=False),
        scratch_types=[pltpu.VMEM((ch,), jnp.int32),
                       pltpu.VMEM((ch,), jnp.int32),
                       pltpu.VMEM((ch, width), jnp.float32),
                       pltpu.VMEM((ch, width), jnp.float32)]
        + [pltpu.SemaphoreType.DMA] * 6,
    )
    return f(table, idx2d)


def _prep_body(xcg_ref, ex_ref, wie_ref, msg0_ref, b_ref):
    ew = jnp.dot(ex_ref[...], wie_ref[...], preferred_element_type=jnp.float32,
                 precision=_PREC)
    msg0_ref[...] = jnp.maximum(xcg_ref[:, :H] + ew, 0.0)
    b_ref[...] = xcg_ref[:, H:] + ew


def _edge_prep(xcg, ex8, wie8):
    E = xcg.shape[0]
    BLK = 8000
    grid = (E // BLK,)
    bs = lambda w: pl.BlockSpec((BLK, w), lambda i: (i, 0))
    return pl.pallas_call(
        _prep_body,
        grid=grid,
        in_specs=[bs(2 * H), bs(8), pl.BlockSpec((8, H), lambda i: (0, 0))],
        out_specs=[bs(H), bs(H)],
        out_shape=[jax.ShapeDtypeStruct((E, H), jnp.float32),
                   jax.ShapeDtypeStruct((E, H), jnp.float32)],
    )(xcg, ex8, wie8)


def _iter_body(g_ref, m_ref, b_ref, w2_ref, o_ref):
    t = g_ref[...] - m_ref[...]
    tw = jnp.dot(t, w2_ref[...], preferred_element_type=jnp.float32,
                 precision=_PREC)
    accum = jnp.concatenate([tw[:, H:], tw[:, :H]], axis=1)
    o_ref[...] = jnp.maximum(b_ref[...] + accum, 0.0)


def _edge_iter(g, msg, b, w2):
    EP = g.shape[0]
    BLK = 5000
    grid = (EP // BLK,)
    bs = pl.BlockSpec((BLK, 2 * H), lambda i: (i, 0))
    return pl.pallas_call(
        _iter_body,
        grid=grid,
        in_specs=[bs, bs, bs, pl.BlockSpec((2 * H, 2 * H), lambda i: (0, 0))],
        out_specs=bs,
        out_shape=jax.ShapeDtypeStruct((EP, 2 * H), jnp.float32),
    )(g, msg, b, w2)


def _final_body(x64_ref, m_ref, gid_ref, wo_ref, bo_ref, sums_ref, cnts_ref):
    i = pl.program_id(0)

    @pl.when(i == 0)
    def _():
        sums_ref[...] = jnp.zeros_like(sums_ref)
        cnts_ref[...] = jnp.zeros_like(cnts_ref)

    xm = jnp.concatenate([x64_ref[...], m_ref[...]], axis=1)
    h = jnp.maximum(
        jnp.dot(xm, wo_ref[...], preferred_element_type=jnp.float32,
                precision=_PREC) + bo_ref[...], 0.0)
    oneh = (gid_ref[...] == jax.lax.broadcasted_iota(jnp.int32, (1, NG), 1)
            ).astype(jnp.float32)
    sums_ref[...] += jax.lax.dot_general(
        oneh, h, (((0,), (0,)), ((), ())), preferred_element_type=jnp.float32,
        precision=_PREC)
    cnts_ref[...] += jnp.sum(oneh, axis=0, keepdims=True)


def _final(x64, m, gid2d, wo128, bo2d):
    Nn = x64.shape[0]
    BLK = 2000
    grid = (Nn // BLK,)
    bs = lambda w: pl.BlockSpec((BLK, w), lambda i: (i, 0))
    return pl.pallas_call(
        _final_body,
        grid=grid,
        in_specs=[bs(H), bs(H), pl.BlockSpec((BLK, 1), lambda i: (i, 0)),
                  pl.BlockSpec((2 * H, H), lambda i: (0, 0)),
                  pl.BlockSpec((1, H), lambda i: (0, 0))],
        out_specs=[pl.BlockSpec((NG, H), lambda i: (0, 0)),
                   pl.BlockSpec((1, NG), lambda i: (0, 0))],
        out_shape=[jax.ShapeDtypeStruct((NG, H), jnp.float32),
                   jax.ShapeDtypeStruct((1, NG), jnp.float32)],
    )(x64, m, gid2d, wo128, bo2d)


def kernel(x, edge_index, edge_x, tree_mess, tree_mess_tgt_nodes, graph_ids,
           W_i, W_h, W_o, b_o):
    n = x.shape[0]
    E = edge_index.shape[1]
    src = edge_index[0].astype(jnp.int32)
    dst = edge_index[1].astype(jnp.int32)
    depth_minus_1 = 3

    node_alpha = jnp.zeros((n, H), x.dtype).at[tree_mess_tgt_nodes].add(tree_mess)
    xp = x @ W_i[:, :x.shape[1]].T
    c = xp + node_alpha @ W_h.T

    ex8 = jnp.pad(edge_x, ((0, 0), (0, 8 - edge_x.shape[1])))
    wie8 = jnp.pad(W_i[:, x.shape[1]:], ((0, 0), (0, 8 - edge_x.shape[1]))).T
    xc = jnp.concatenate([xp, c], axis=1)
    xcg = _sc_gather(xc, src.reshape(E // 320, 320), 2 * H, 320)
    msg, b = _edge_prep(xcg, ex8, wie8)

    w2 = jnp.kron(jnp.eye(2, dtype=jnp.float32), W_h.T)
    dst2d = dst.reshape(E // 640, 640)
    msg_p = msg.reshape(E // 2, 2 * H)
    b_p = b.reshape(E // 2, 2 * H)
    for _ in range(depth_minus_1):
        ns = jax.ops.segment_sum(msg_p.reshape(E, H), dst, num_segments=n)
        g = _sc_gather(ns, dst2d, H, 640).reshape(E // 2, 2 * H)
        msg_p = _edge_iter(g, msg_p, b_p, w2)

    ns = jax.ops.segment_sum(msg_p.reshape(E, H), dst, num_segments=n)
    m = ns + node_alpha

    x64 = jnp.pad(x, ((0, 0), (0, H - x.shape[1])))
    wo128 = jnp.concatenate(
        [jnp.pad(W_o[:, :x.shape[1]], ((0, 0), (0, H - x.shape[1]))),
         W_o[:, x.shape[1]:]], axis=1).T
    sums, cnts = _final(x64, m, graph_ids.reshape(n, 1).astype(jnp.int32),
                        wo128, b_o.reshape(1, H))
    return sums / jnp.maximum(cnts.reshape(NG, 1), 1.0)


# speedup vs baseline: 2.2637x; 1.1276x over previous
import functools

import jax
import jax.numpy as jnp
from jax import lax
from jax.experimental import pallas as pl
from jax.experimental.pallas import tpu as pltpu
from jax.experimental.pallas import tpu_sc as plsc

H = 64
NG = 256
NC, NS = 2, 16
NW = NC * NS
_PREC = jax.lax.Precision.HIGHEST


def _sc_gather(table, idx2d, width, ch):
    nch = idx2d.shape[0]
    e_out = nch * ch
    mesh = plsc.VectorSubcoreMesh(core_axis_name="c", subcore_axis_name="s")

    def body(table_hbm, idx_hbm, out_hbm, ib0, ib1, rb0, rb1,
             isem0, isem1, gsem0, gsem1, osem0, osem1):
        wid = lax.axis_index("s") * NC + lax.axis_index("c")
        ibufs, rbufs = (ib0, ib1), (rb0, rb1)
        isems, gsems, osems = (isem0, isem1), (gsem0, gsem1), (osem0, osem1)

        for b in (0, 1):
            ch0 = wid + NW * b

            @pl.when(ch0 < nch)
            def _():
                pltpu.make_async_copy(idx_hbm.at[ch0], ibufs[b], isems[b]).start()

        @pl.loop(wid, nch, step=2 * NW)
        def _(g):
            for b in (0, 1):
                cno = g + NW * b

                @pl.when(cno < nch)
                def _():
                    @pl.when(cno >= wid + 2 * NW)
                    def _():
                        pltpu.make_async_copy(
                            rbufs[b], out_hbm.at[pl.ds((cno - 2 * NW) * ch, ch)],
                            osems[b]).wait()

                    pltpu.make_async_copy(idx_hbm.at[cno], ibufs[b],
                                          isems[b]).wait()
                    pltpu.make_async_copy(table_hbm.at[ibufs[b]], rbufs[b],
                                          gsems[b]).start()
                    pltpu.make_async_copy(table_hbm.at[ibufs[b]], rbufs[b],
                                          gsems[b]).wait()

                    @pl.when(cno + 2 * NW < nch)
                    def _():
                        pltpu.make_async_copy(idx_hbm.at[cno + 2 * NW],
                                              ibufs[b], isems[b]).start()

                    pltpu.make_async_copy(
                        rbufs[b], out_hbm.at[pl.ds(cno * ch, ch)],
                        osems[b]).start()

        for b in (0, 1):
            last = ((nch - 1 - wid - NW * b) // (2 * NW)) * 2 * NW + wid + NW * b

            @pl.when((last >= 0) & (last < nch))
            def _():
                pltpu.make_async_copy(
                    rbufs[b], out_hbm.at[pl.ds(last * ch, ch)], osems[b]).wait()

    f = pl.kernel(
        body, mesh=mesh,
        out_type=jax.ShapeDtypeStruct((e_out, width), jnp.float32),
        compiler_params=pltpu.CompilerParams(use_tc_tiling_on_sc=False),
        scratch_types=[pltpu.VMEM((ch,), jnp.int32),
                       pltpu.VMEM((ch,), jnp.int32),
                       pltpu.VMEM((ch, width), jnp.float32),
                       pltpu.VMEM((ch, width), jnp.float32)]
        + [pltpu.SemaphoreType.DMA] * 6,
    )
    return f(table, idx2d)


QUARTER = 12544
_SHARE = QUARTER // NS


def _sc_segment_sum(data, idx2d, ch):
    nch = idx2d.shape[0]
    mesh = plsc.VectorSubcoreMesh(core_axis_name="c", subcore_axis_name="s")

    def body(data_hbm, idx_hbm, out_hbm, table, ib0, ib1, db0, db1,
             isem0, isem1, dsem0, dsem1):
        core = lax.axis_index("c")
        tid = lax.axis_index("s")
        ibufs, dbufs = (ib0, ib1), (db0, db1)
        isems, dsems = (isem0, isem1), (dsem0, dsem1)
        zvec = jnp.zeros((16,), jnp.float32)

        def start_loads(cno, b):
            pltpu.make_async_copy(idx_hbm.at[cno], ibufs[b], isems[b]).start()
            pltpu.make_async_copy(data_hbm.at[pl.ds(cno * ch, ch)], dbufs[b],
                                  dsems[b]).start()

        for p in (0, 1):
            lo = (2 * p + core) * QUARTER

            @pl.loop(0, ch)
            def _(r):
                for c0 in range(0, H, 16):
                    db0[r, pl.ds(c0, 16)] = zvec

            nz = _SHARE // ch
            for z in range(nz):
                pltpu.sync_copy(
                    db0, table.at[pl.ds(tid * _SHARE + z * ch, ch)])
            rem = _SHARE - nz * ch
            if rem:
                pltpu.sync_copy(
                    db0.at[pl.ds(0, rem)],
                    table.at[pl.ds(tid * _SHARE + nz * ch, rem)])

            @pl.when(tid == 0)
            def _():
                pltpu.sync_copy(db0.at[pl.ds(0, 8)],
                                table.at[pl.ds(QUARTER, 8)])

            plsc.subcore_barrier()

            for b in (0, 1):
                cno = tid + NS * b

                @pl.when(cno < nch)
                def _():
                    start_loads(cno, b)

            @pl.loop(tid, nch, step=2 * NS)
            def _(g):
                for b in (0, 1):
                    cno = g + NS * b

                    @pl.when(cno < nch)
                    def _():
                        pltpu.make_async_copy(idx_hbm.at[cno], ibufs[b],
                                              isems[b]).wait()
                        pltpu.make_async_copy(
                            data_hbm.at[pl.ds(cno * ch, ch)], dbufs[b],
                            dsems[b]).wait()

                        @pl.loop(0, ch, step=16)
                        def _(j):
                            v = ibufs[b][pl.ds(j, 16)]
                            ok = (v >= lo) & (v < lo + QUARTER)
                            ibufs[b][pl.ds(j, 16)] = jnp.where(
                                ok, v - lo, QUARTER)

                        pltpu.sync_copy(dbufs[b], table.at[ibufs[b]],
                                        add=True)

                        @pl.when(cno + 2 * NS < nch)
                        def _():
                            start_loads(cno + 2 * NS, b)

            plsc.subcore_barrier()

            for z in range((_SHARE + ch - 1) // ch):
                rows = min(ch, _SHARE - z * ch)
                start = tid * _SHARE + z * ch
                pltpu.sync_copy(table.at[pl.ds(start, rows)],
                                out_hbm.at[pl.ds(lo + start, rows)])

            if p == 0:
                plsc.subcore_barrier()

    f = pl.kernel(
        body, mesh=mesh,
        out_type=jax.ShapeDtypeStruct((4 * QUARTER, H), jnp.float32),
        compiler_params=pltpu.CompilerParams(use_tc_tiling_on_sc=False),
        scratch_types=[pltpu.VMEM_SHARED((QUARTER + 8, H), jnp.float32),
                       pltpu.VMEM((ch,), jnp.int32),
                       pltpu.VMEM((ch,), jnp.int32),
                       pltpu.VMEM((ch, H), jnp.float32),
                       pltpu.VMEM((ch, H), jnp.float32)]
        + [pltpu.SemaphoreType.DMA] * 4,
    )
    return f(data, idx2d)


def _prep_body(xcg_ref, ex_ref, wie_ref, msg0_ref, b_ref):
    ew = jnp.dot(ex_ref[...], wie_ref[...], preferred_element_type=jnp.float32,
                 precision=_PREC)
    msg0_ref[...] = jnp.maximum(xcg_ref[:, :H] + ew, 0.0)
    b_ref[...] = xcg_ref[:, H:] + ew


def _edge_prep(xcg, ex8, wie8):
    E = xcg.shape[0]
    BLK = 8000
    grid = (E // BLK,)
    bs = lambda w: pl.BlockSpec((BLK, w), lambda i: (i, 0))
    return pl.pallas_call(
        _prep_body,
        grid=grid,
        in_specs=[bs(2 * H), bs(8), pl.BlockSpec((8, H), lambda i: (0, 0))],
        out_specs=[bs(H), bs(H)],
        out_shape=[jax.ShapeDtypeStruct((E, H), jnp.float32),
                   jax.ShapeDtypeStruct((E, H), jnp.float32)],
    )(xcg, ex8, wie8)


def _iter_body(g_ref, m_ref, b_ref, w2_ref, o_ref):
    t = g_ref[...] - m_ref[...]
    tw = jnp.dot(t, w2_ref[...], preferred_element_type=jnp.float32,
                 precision=_PREC)
    accum = jnp.concatenate([tw[:, H:], tw[:, :H]], axis=1)
    o_ref[...] = jnp.maximum(b_ref[...] + accum, 0.0)


def _edge_iter(g, msg, b, w2):
    EP = g.shape[0]
    BLK = 5000
    grid = (EP // BLK,)
    bs = pl.BlockSpec((BLK, 2 * H), lambda i: (i, 0))
    return pl.pallas_call(
        _iter_body,
        grid=grid,
        in_specs=[bs, bs, bs, pl.BlockSpec((2 * H, 2 * H), lambda i: (0, 0))],
        out_specs=bs,
        out_shape=jax.ShapeDtypeStruct((EP, 2 * H), jnp.float32),
    )(g, msg, b, w2)


def _final_body(x64_ref, m_ref, gid_ref, wo_ref, bo_ref, sums_ref, cnts_ref):
    i = pl.program_id(0)

    @pl.when(i == 0)
    def _():
        sums_ref[...] = jnp.zeros_like(sums_ref)
        cnts_ref[...] = jnp.zeros_like(cnts_ref)

    xm = jnp.concatenate([x64_ref[...], m_ref[...]], axis=1)
    h = jnp.maximum(
        jnp.dot(xm, wo_ref[...], preferred_element_type=jnp.float32,
                precision=_PREC) + bo_ref[...], 0.0)
    oneh = (gid_ref[...] == jax.lax.broadcasted_iota(jnp.int32, (1, NG), 1)
            ).astype(jnp.float32)
    sums_ref[...] += jax.lax.dot_general(
        oneh, h, (((0,), (0,)), ((), ())), preferred_element_type=jnp.float32,
        precision=_PREC)
    cnts_ref[...] += jnp.sum(oneh, axis=0, keepdims=True)


def _final(x64, m, gid2d, wo128, bo2d):
    Nn = x64.shape[0]
    BLK = 2000
    grid = (Nn // BLK,)
    bs = lambda w: pl.BlockSpec((BLK, w), lambda i: (i, 0))
    return pl.pallas_call(
        _final_body,
        grid=grid,
        in_specs=[bs(H), bs(H), pl.BlockSpec((BLK, 1), lambda i: (i, 0)),
                  pl.BlockSpec((2 * H, H), lambda i: (0, 0)),
                  pl.BlockSpec((1, H), lambda i: (0, 0))],
        out_specs=[pl.BlockSpec((NG, H), lambda i: (0, 0)),
                   pl.BlockSpec((1, NG), lambda i: (0, 0))],
        out_shape=[jax.ShapeDtypeStruct((NG, H), jnp.float32),
                   jax.ShapeDtypeStruct((1, NG), jnp.float32)],
    )(x64, m, gid2d, wo128, bo2d)


def kernel(x, edge_index, edge_x, tree_mess, tree_mess_tgt_nodes, graph_ids,
           W_i, W_h, W_o, b_o):
    n = x.shape[0]
    E = edge_index.shape[1]
    src = edge_index[0].astype(jnp.int32)
    dst = edge_index[1].astype(jnp.int32)
    depth_minus_1 = 3

    na_p = _sc_segment_sum(
        tree_mess, tree_mess_tgt_nodes.astype(jnp.int32).reshape(-1, 400), 400)
    node_alpha = na_p[:n]
    xp = x @ W_i[:, :x.shape[1]].T
    c = xp + node_alpha @ W_h.T

    ex8 = jnp.pad(edge_x, ((0, 0), (0, 8 - edge_x.shape[1])))
    wie8 = jnp.pad(W_i[:, x.shape[1]:], ((0, 0), (0, 8 - edge_x.shape[1]))).T
    xc = jnp.concatenate([xp, c], axis=1)
    xcg = _sc_gather(xc, src.reshape(E // 320, 320), 2 * H, 320)
    msg, b = _edge_prep(xcg, ex8, wie8)

    w2 = jnp.kron(jnp.eye(2, dtype=jnp.float32), W_h.T)
    dst2d = dst.reshape(E // 640, 640)
    dst_sc = dst.reshape(E // 400, 400)
    msg_p = msg.reshape(E // 2, 2 * H)
    b_p = b.reshape(E // 2, 2 * H)
    for _ in range(depth_minus_1):
        ns_p = _sc_segment_sum(msg_p.reshape(E, H), dst_sc, 400)
        g = _sc_gather(ns_p, dst2d, H, 640).reshape(E // 2, 2 * H)
        msg_p = _edge_iter(g, msg_p, b_p, w2)

    ns_p = _sc_segment_sum(msg_p.reshape(E, H), dst_sc, 400)
    m = (ns_p + na_p)[:n]

    x64 = jnp.pad(x, ((0, 0), (0, H - x.shape[1])))
    wo128 = jnp.concatenate(
        [jnp.pad(W_o[:, :x.shape[1]], ((0, 0), (0, H - x.shape[1]))),
         W_o[:, x.shape[1]:]], axis=1).T
    sums, cnts = _final(x64, m, graph_ids.reshape(n, 1).astype(jnp.int32),
                        wo128, b_o.reshape(1, H))
    return sums / jnp.maximum(cnts.reshape(NG, 1), 1.0)


# speedup vs baseline: 3.7223x; 1.6443x over previous
import functools

import jax
import jax.numpy as jnp
from jax import lax
from jax.experimental import pallas as pl
from jax.experimental.pallas import tpu as pltpu
from jax.experimental.pallas import tpu_sc as plsc

H = 64
NG = 256
NC, NS = 2, 16
NW = NC * NS
_PREC = jax.lax.Precision.HIGHEST


def _sc_gather(table, idx2d, width, ch):
    nch = idx2d.shape[0]
    e_out = nch * ch
    mesh = plsc.VectorSubcoreMesh(core_axis_name="c", subcore_axis_name="s")

    def body(table_hbm, idx_hbm, out_hbm, ib0, ib1, rb0, rb1,
             isem0, isem1, gsem0, gsem1, osem0, osem1):
        wid = lax.axis_index("s") * NC + lax.axis_index("c")
        ibufs, rbufs = (ib0, ib1), (rb0, rb1)
        isems, gsems, osems = (isem0, isem1), (gsem0, gsem1), (osem0, osem1)

        for b in (0, 1):
            ch0 = wid + NW * b

            @pl.when(ch0 < nch)
            def _():
                pltpu.make_async_copy(idx_hbm.at[ch0], ibufs[b], isems[b]).start()

        @pl.loop(wid, nch, step=2 * NW)
        def _(g):
            for b in (0, 1):
                cno = g + NW * b

                @pl.when(cno < nch)
                def _():
                    @pl.when(cno >= wid + 2 * NW)
                    def _():
                        pltpu.make_async_copy(
                            rbufs[b], out_hbm.at[pl.ds((cno - 2 * NW) * ch, ch)],
                            osems[b]).wait()

                    pltpu.make_async_copy(idx_hbm.at[cno], ibufs[b],
                                          isems[b]).wait()
                    pltpu.make_async_copy(table_hbm.at[ibufs[b]], rbufs[b],
                                          gsems[b]).start()
                    pltpu.make_async_copy(table_hbm.at[ibufs[b]], rbufs[b],
                                          gsems[b]).wait()

                    @pl.when(cno + 2 * NW < nch)
                    def _():
                        pltpu.make_async_copy(idx_hbm.at[cno + 2 * NW],
                                              ibufs[b], isems[b]).start()

                    pltpu.make_async_copy(
                        rbufs[b], out_hbm.at[pl.ds(cno * ch, ch)],
                        osems[b]).start()

        for b in (0, 1):
            last = ((nch - 1 - wid - NW * b) // (2 * NW)) * 2 * NW + wid + NW * b

            @pl.when((last >= 0) & (last < nch))
            def _():
                pltpu.make_async_copy(
                    rbufs[b], out_hbm.at[pl.ds(last * ch, ch)], osems[b]).wait()

    f = pl.kernel(
        body, mesh=mesh,
        out_type=jax.ShapeDtypeStruct((e_out, width), jnp.float32),
        compiler_params=pltpu.CompilerParams(use_tc_tiling_on_sc=False),
        scratch_types=[pltpu.VMEM((ch,), jnp.int32),
                       pltpu.VMEM((ch,), jnp.int32),
                       pltpu.VMEM((ch, width), jnp.float32),
                       pltpu.VMEM((ch, width), jnp.float32)]
        + [pltpu.SemaphoreType.DMA] * 6,
    )
    return f(table, idx2d)


HALF = 25088
_SHARE = HALF // NS


def _sc_segment_sum(data, idx2d, ch):
    nch = idx2d.shape[0]
    mesh = plsc.VectorSubcoreMesh(core_axis_name="c", subcore_axis_name="s")

    def body(data_hbm, idx_hbm, out_hbm, table, ib0, ib1, db0, db1,
             isem0, isem1, dsem0, dsem1):
        core = lax.axis_index("c")
        tid = lax.axis_index("s")
        ibufs, dbufs = (ib0, ib1), (db0, db1)
        isems, dsems = (isem0, isem1), (dsem0, dsem1)
        zvec = jnp.zeros((16,), jnp.float32)
        lo = core * HALF

        def start_loads(cno, b):
            pltpu.make_async_copy(idx_hbm.at[cno], ibufs[b], isems[b]).start()
            pltpu.make_async_copy(data_hbm.at[pl.ds(cno * ch, ch)], dbufs[b],
                                  dsems[b]).start()

        @pl.loop(0, ch)
        def _(r):
            for c0 in range(0, H, 16):
                db0[r, pl.ds(c0, 16)] = zvec

        nz = _SHARE // ch
        for z in range(nz):
            pltpu.sync_copy(db0, table.at[pl.ds(tid * _SHARE + z * ch, ch)])
        rem = _SHARE - nz * ch
        if rem:
            pltpu.sync_copy(db0.at[pl.ds(0, rem)],
                            table.at[pl.ds(tid * _SHARE + nz * ch, rem)])

        @pl.when(tid == 0)
        def _():
            pltpu.sync_copy(db0.at[pl.ds(0, 8)], table.at[pl.ds(HALF, 8)])

        plsc.subcore_barrier()

        for b in (0, 1):
            cno = tid + NS * b

            @pl.when(cno < nch)
            def _():
                start_loads(cno, b)

        @pl.loop(tid, nch, step=2 * NS)
        def _(g):
            for b in (0, 1):
                cno = g + NS * b

                @pl.when(cno < nch)
                def _():
                    pltpu.make_async_copy(idx_hbm.at[cno], ibufs[b],
                                          isems[b]).wait()
                    pltpu.make_async_copy(data_hbm.at[pl.ds(cno * ch, ch)],
                                          dbufs[b], dsems[b]).wait()

                    @pl.loop(0, ch, step=16)
                    def _(j):
                        v = ibufs[b][pl.ds(j, 16)]
                        ok = (v >= lo) & (v < lo + HALF)
                        ibufs[b][pl.ds(j, 16)] = jnp.where(ok, v - lo, HALF)

                    pltpu.sync_copy(dbufs[b], table.at[ibufs[b]], add=True)

                    @pl.when(cno + 2 * NS < nch)
                    def _():
                        start_loads(cno + 2 * NS, b)

        plsc.subcore_barrier()

        for z in range((_SHARE + ch - 1) // ch):
            rows = min(ch, _SHARE - z * ch)
            start = tid * _SHARE + z * ch
            pltpu.sync_copy(table.at[pl.ds(start, rows)],
                            out_hbm.at[pl.ds(lo + start, rows)])

    f = pl.kernel(
        body, mesh=mesh,
        out_type=jax.ShapeDtypeStruct((2 * HALF, H), jnp.float32),
        compiler_params=pltpu.CompilerParams(use_tc_tiling_on_sc=False),
        scratch_types=[pltpu.VMEM_SHARED((HALF + 8, H), jnp.float32),
                       pltpu.VMEM((ch,), jnp.int32),
                       pltpu.VMEM((ch,), jnp.int32),
                       pltpu.VMEM((ch, H), jnp.float32),
                       pltpu.VMEM((ch, H), jnp.float32)]
        + [pltpu.SemaphoreType.DMA] * 4,
    )
    return f(data, idx2d)


def _prep_body(xcg_ref, ex_ref, wie_ref, msg0_ref, b_ref):
    ew = jnp.dot(ex_ref[...], wie_ref[...], preferred_element_type=jnp.float32,
                 precision=_PREC)
    msg0_ref[...] = jnp.maximum(xcg_ref[:, :H] + ew, 0.0)
    b_ref[...] = xcg_ref[:, H:] + ew


def _edge_prep(xcg, ex8, wie8):
    E = xcg.shape[0]
    BLK = 8000
    grid = (E // BLK,)
    bs = lambda w: pl.BlockSpec((BLK, w), lambda i: (i, 0))
    return pl.pallas_call(
        _prep_body,
        grid=grid,
        in_specs=[bs(2 * H), bs(8), pl.BlockSpec((8, H), lambda i: (0, 0))],
        out_specs=[bs(H), bs(H)],
        out_shape=[jax.ShapeDtypeStruct((E, H), jnp.float32),
                   jax.ShapeDtypeStruct((E, H), jnp.float32)],
    )(xcg, ex8, wie8)


def _iter_body(g_ref, m_ref, b_ref, w2_ref, o_ref):
    t = g_ref[...] - m_ref[...]
    tw = jnp.dot(t, w2_ref[...], preferred_element_type=jnp.float32,
                 precision=_PREC)
    accum = jnp.concatenate([tw[:, H:], tw[:, :H]], axis=1)
    o_ref[...] = jnp.maximum(b_ref[...] + accum, 0.0)


def _edge_iter(g, msg, b, w2):
    EP = g.shape[0]
    BLK = 5000
    grid = (EP // BLK,)
    bs = pl.BlockSpec((BLK, 2 * H), lambda i: (i, 0))
    return pl.pallas_call(
        _iter_body,
        grid=grid,
        in_specs=[bs, bs, bs, pl.BlockSpec((2 * H, 2 * H), lambda i: (0, 0))],
        out_specs=bs,
        out_shape=jax.ShapeDtypeStruct((EP, 2 * H), jnp.float32),
    )(g, msg, b, w2)


def _final_body(x64_ref, m_ref, gid_ref, wo_ref, bo_ref, sums_ref, cnts_ref):
    i = pl.program_id(0)

    @pl.when(i == 0)
    def _():
        sums_ref[...] = jnp.zeros_like(sums_ref)
        cnts_ref[...] = jnp.zeros_like(cnts_ref)

    xm = jnp.concatenate([x64_ref[...], m_ref[...]], axis=1)
    h = jnp.maximum(
        jnp.dot(xm, wo_ref[...], preferred_element_type=jnp.float32,
                precision=_PREC) + bo_ref[...], 0.0)
    oneh = (gid_ref[...] == jax.lax.broadcasted_iota(jnp.int32, (1, NG), 1)
            ).astype(jnp.float32)
    sums_ref[...] += jax.lax.dot_general(
        oneh, h, (((0,), (0,)), ((), ())), preferred_element_type=jnp.float32,
        precision=_PREC)
    cnts_ref[...] += jnp.sum(oneh, axis=0, keepdims=True)


def _final(x64, m, gid2d, wo128, bo2d):
    Nn = x64.shape[0]
    BLK = 2000
    grid = (Nn // BLK,)
    bs = lambda w: pl.BlockSpec((BLK, w), lambda i: (i, 0))
    return pl.pallas_call(
        _final_body,
        grid=grid,
        in_specs=[bs(H), bs(H), pl.BlockSpec((BLK, 1), lambda i: (i, 0)),
                  pl.BlockSpec((2 * H, H), lambda i: (0, 0)),
                  pl.BlockSpec((1, H), lambda i: (0, 0))],
        out_specs=[pl.BlockSpec((NG, H), lambda i: (0, 0)),
                   pl.BlockSpec((1, NG), lambda i: (0, 0))],
        out_shape=[jax.ShapeDtypeStruct((NG, H), jnp.float32),
                   jax.ShapeDtypeStruct((1, NG), jnp.float32)],
    )(x64, m, gid2d, wo128, bo2d)


def kernel(x, edge_index, edge_x, tree_mess, tree_mess_tgt_nodes, graph_ids,
           W_i, W_h, W_o, b_o):
    n = x.shape[0]
    E = edge_index.shape[1]
    src = edge_index[0].astype(jnp.int32)
    dst = edge_index[1].astype(jnp.int32)
    depth_minus_1 = 3

    na_p = _sc_segment_sum(
        tree_mess, tree_mess_tgt_nodes.astype(jnp.int32).reshape(-1, 160), 160)
    node_alpha = na_p[:n]
    xp = x @ W_i[:, :x.shape[1]].T
    c = xp + node_alpha @ W_h.T

    ex8 = jnp.pad(edge_x, ((0, 0), (0, 8 - edge_x.shape[1])))
    wie8 = jnp.pad(W_i[:, x.shape[1]:], ((0, 0), (0, 8 - edge_x.shape[1]))).T
    xc = jnp.concatenate([xp, c], axis=1)
    xcg = _sc_gather(xc, src.reshape(E // 320, 320), 2 * H, 320)
    msg, b = _edge_prep(xcg, ex8, wie8)

    w2 = jnp.kron(jnp.eye(2, dtype=jnp.float32), W_h.T)
    dst2d = dst.reshape(E // 640, 640)
    dst_sc = dst.reshape(E // 160, 160)
    msg_p = msg.reshape(E // 2, 2 * H)
    b_p = b.reshape(E // 2, 2 * H)
    for _ in range(depth_minus_1):
        ns_p = _sc_segment_sum(msg_p.reshape(E, H), dst_sc, 160)
        g = _sc_gather(ns_p, dst2d, H, 640).reshape(E // 2, 2 * H)
        msg_p = _edge_iter(g, msg_p, b_p, w2)

    ns_p = _sc_segment_sum(msg_p.reshape(E, H), dst_sc, 160)
    m = (ns_p + na_p)[:n]

    x64 = jnp.pad(x, ((0, 0), (0, H - x.shape[1])))
    wo128 = jnp.concatenate(
        [jnp.pad(W_o[:, :x.shape[1]], ((0, 0), (0, H - x.shape[1]))),
         W_o[:, x.shape[1]:]], axis=1).T
    sums, cnts = _final(x64, m, graph_ids.reshape(n, 1).astype(jnp.int32),
                        wo128, b_o.reshape(1, H))
    return sums / jnp.maximum(cnts.reshape(NG, 1), 1.0)


# speedup vs baseline: 3.9492x; 1.0610x over previous
import functools

import jax
import jax.numpy as jnp
from jax import lax
from jax.experimental import pallas as pl
from jax.experimental.pallas import tpu as pltpu
from jax.experimental.pallas import tpu_sc as plsc

H = 64
NG = 256
NC, NS = 2, 16
NW = NC * NS
_PREC = jax.lax.Precision.HIGHEST


def _sc_gather(table, idx2d, width, ch):
    nch = idx2d.shape[0]
    e_out = nch * ch
    mesh = plsc.VectorSubcoreMesh(core_axis_name="c", subcore_axis_name="s")

    def body(table_hbm, idx_hbm, out_hbm, ib0, ib1, rb0, rb1,
             isem0, isem1, gsem0, gsem1, osem0, osem1):
        wid = lax.axis_index("s") * NC + lax.axis_index("c")
        ibufs, rbufs = (ib0, ib1), (rb0, rb1)
        isems, gsems, osems = (isem0, isem1), (gsem0, gsem1), (osem0, osem1)

        for b in (0, 1):
            ch0 = wid + NW * b

            @pl.when(ch0 < nch)
            def _():
                pltpu.make_async_copy(idx_hbm.at[ch0], ibufs[b], isems[b]).start()

        @pl.loop(wid, nch, step=2 * NW)
        def _(g):
            for b in (0, 1):
                cno = g + NW * b

                @pl.when(cno < nch)
                def _():
                    @pl.when(cno >= wid + 2 * NW)
                    def _():
                        pltpu.make_async_copy(
                            rbufs[b], out_hbm.at[pl.ds((cno - 2 * NW) * ch, ch)],
                            osems[b]).wait()

                    pltpu.make_async_copy(idx_hbm.at[cno], ibufs[b],
                                          isems[b]).wait()
                    pltpu.make_async_copy(table_hbm.at[ibufs[b]], rbufs[b],
                                          gsems[b]).start()
                    pltpu.make_async_copy(table_hbm.at[ibufs[b]], rbufs[b],
                                          gsems[b]).wait()

                    @pl.when(cno + 2 * NW < nch)
                    def _():
                        pltpu.make_async_copy(idx_hbm.at[cno + 2 * NW],
                                              ibufs[b], isems[b]).start()

                    pltpu.make_async_copy(
                        rbufs[b], out_hbm.at[pl.ds(cno * ch, ch)],
                        osems[b]).start()

        for b in (0, 1):
            last = ((nch - 1 - wid - NW * b) // (2 * NW)) * 2 * NW + wid + NW * b

            @pl.when((last >= 0) & (last < nch))
            def _():
                pltpu.make_async_copy(
                    rbufs[b], out_hbm.at[pl.ds(last * ch, ch)], osems[b]).wait()

    f = pl.kernel(
        body, mesh=mesh,
        out_type=jax.ShapeDtypeStruct((e_out, width), jnp.float32),
        compiler_params=pltpu.CompilerParams(use_tc_tiling_on_sc=False),
        scratch_types=[pltpu.VMEM((ch,), jnp.int32),
                       pltpu.VMEM((ch,), jnp.int32),
                       pltpu.VMEM((ch, width), jnp.float32),
                       pltpu.VMEM((ch, width), jnp.float32)]
        + [pltpu.SemaphoreType.DMA] * 6,
    )
    return f(table, idx2d)


HALF = 25088
_SHARE = HALF // NS


def _sc_segment_sum(data, idx2d, ch):
    nch = idx2d.shape[0]
    mesh = plsc.VectorSubcoreMesh(core_axis_name="c", subcore_axis_name="s")

    def body(data_hbm, idx_hbm, out_hbm, table, ib0, ib1, db0, db1,
             isem0, isem1, dsem0, dsem1):
        core = lax.axis_index("c")
        tid = lax.axis_index("s")
        ibufs, dbufs = (ib0, ib1), (db0, db1)
        isems, dsems = (isem0, isem1), (dsem0, dsem1)
        zvec = jnp.zeros((16,), jnp.float32)
        lo = core * HALF

        def start_loads(cno, b):
            pltpu.make_async_copy(idx_hbm.at[cno], ibufs[b], isems[b]).start()
            pltpu.make_async_copy(data_hbm.at[pl.ds(cno * ch, ch)], dbufs[b],
                                  dsems[b]).start()

        @pl.loop(0, ch)
        def _(r):
            for c0 in range(0, H, 16):
                db0[r, pl.ds(c0, 16)] = zvec

        nz = _SHARE // ch
        for z in range(nz):
            pltpu.sync_copy(db0, table.at[pl.ds(tid * _SHARE + z * ch, ch)])
        rem = _SHARE - nz * ch
        if rem:
            pltpu.sync_copy(db0.at[pl.ds(0, rem)],
                            table.at[pl.ds(tid * _SHARE + nz * ch, rem)])

        @pl.when(tid == 0)
        def _():
            pltpu.sync_copy(db0.at[pl.ds(0, 8)], table.at[pl.ds(HALF, 8)])

        plsc.subcore_barrier()

        for b in (0, 1):
            cno = tid + NS * b

            @pl.when(cno < nch)
            def _():
                start_loads(cno, b)

        @pl.loop(tid, nch, step=2 * NS)
        def _(g):
            for b in (0, 1):
                cno = g + NS * b

                @pl.when(cno < nch)
                def _():
                    pltpu.make_async_copy(idx_hbm.at[cno], ibufs[b],
                                          isems[b]).wait()
                    pltpu.make_async_copy(data_hbm.at[pl.ds(cno * ch, ch)],
                                          dbufs[b], dsems[b]).wait()

                    @pl.loop(0, ch, step=16)
                    def _(j):
                        v = ibufs[b][pl.ds(j, 16)]
                        ok = (v >= lo) & (v < lo + HALF)
                        ibufs[b][pl.ds(j, 16)] = jnp.where(ok, v - lo, HALF)

                    pltpu.sync_copy(dbufs[b], table.at[ibufs[b]], add=True)

                    @pl.when(cno + 2 * NS < nch)
                    def _():
                        start_loads(cno + 2 * NS, b)

        plsc.subcore_barrier()

        for z in range((_SHARE + ch - 1) // ch):
            rows = min(ch, _SHARE - z * ch)
            start = tid * _SHARE + z * ch
            pltpu.sync_copy(table.at[pl.ds(start, rows)],
                            out_hbm.at[pl.ds(lo + start, rows)])

    f = pl.kernel(
        body, mesh=mesh,
        out_type=jax.ShapeDtypeStruct((2 * HALF, H), jnp.float32),
        compiler_params=pltpu.CompilerParams(use_tc_tiling_on_sc=False),
        scratch_types=[pltpu.VMEM_SHARED((HALF + 8, H), jnp.float32),
                       pltpu.VMEM((ch,), jnp.int32),
                       pltpu.VMEM((ch,), jnp.int32),
                       pltpu.VMEM((ch, H), jnp.float32),
                       pltpu.VMEM((ch, H), jnp.float32)]
        + [pltpu.SemaphoreType.DMA] * 4,
    )
    return f(data, idx2d)


def _prep_body(xcg_ref, code_ref, lut_ref, msg0_ref, b_ref):
    oneh = (code_ref[...] == jax.lax.broadcasted_iota(jnp.int32, (1, 32), 1)
            ).astype(jnp.float32)
    ew = jnp.dot(oneh, lut_ref[...], preferred_element_type=jnp.float32,
                 precision=_PREC)
    msg0_ref[...] = jnp.maximum(xcg_ref[:, :H] + ew, 0.0)
    b_ref[...] = xcg_ref[:, H:] + ew


def _edge_prep(xcg, code, lut):
    E = xcg.shape[0]
    BLK = 8000
    grid = (E // BLK,)
    bs = lambda w: pl.BlockSpec((BLK, w), lambda i: (i, 0))
    return pl.pallas_call(
        _prep_body,
        grid=grid,
        in_specs=[bs(2 * H), pl.BlockSpec((BLK, 1), lambda i: (i, 0)),
                  pl.BlockSpec((32, H), lambda i: (0, 0))],
        out_specs=[bs(H), bs(H)],
        out_shape=[jax.ShapeDtypeStruct((E, H), jnp.float32),
                   jax.ShapeDtypeStruct((E, H), jnp.float32)],
    )(xcg, code, lut)


def _iter_body(g_ref, m_ref, b_ref, w2_ref, o_ref):
    t = g_ref[...] - m_ref[...]
    tw = jnp.dot(t, w2_ref[...], preferred_element_type=jnp.float32,
                 precision=_PREC)
    accum = jnp.concatenate([tw[:, H:], tw[:, :H]], axis=1)
    o_ref[...] = jnp.maximum(b_ref[...] + accum, 0.0)


def _edge_iter(g, msg, b, w2):
    EP = g.shape[0]
    BLK = 8000
    grid = (EP // BLK,)
    bs = pl.BlockSpec((BLK, 2 * H), lambda i: (i, 0))
    return pl.pallas_call(
        _iter_body,
        grid=grid,
        in_specs=[bs, bs, bs, pl.BlockSpec((2 * H, 2 * H), lambda i: (0, 0))],
        out_specs=bs,
        out_shape=jax.ShapeDtypeStruct((EP, 2 * H), jnp.float32),
    )(g, msg, b, w2)


def _final_body(x64_ref, m_ref, gid_ref, wo_ref, bo_ref, sums_ref, cnts_ref):
    i = pl.program_id(0)

    @pl.when(i == 0)
    def _():
        sums_ref[...] = jnp.zeros_like(sums_ref)
        cnts_ref[...] = jnp.zeros_like(cnts_ref)

    xm = jnp.concatenate([x64_ref[...], m_ref[...]], axis=1)
    h = jnp.maximum(
        jnp.dot(xm, wo_ref[...], preferred_element_type=jnp.float32,
                precision=_PREC) + bo_ref[...], 0.0)
    oneh = (gid_ref[...] == jax.lax.broadcasted_iota(jnp.int32, (1, NG), 1)
            ).astype(jnp.float32)
    sums_ref[...] += jax.lax.dot_general(
        oneh, h, (((0,), (0,)), ((), ())), preferred_element_type=jnp.float32,
        precision=_PREC)
    cnts_ref[...] += jnp.sum(oneh, axis=0, keepdims=True)


def _final(x64, m, gid2d, wo128, bo2d):
    Nn = x64.shape[0]
    BLK = 2000
    grid = (Nn // BLK,)
    bs = lambda w: pl.BlockSpec((BLK, w), lambda i: (i, 0))
    return pl.pallas_call(
        _final_body,
        grid=grid,
        in_specs=[bs(H), bs(H), pl.BlockSpec((BLK, 1), lambda i: (i, 0)),
                  pl.BlockSpec((2 * H, H), lambda i: (0, 0)),
                  pl.BlockSpec((1, H), lambda i: (0, 0))],
        out_specs=[pl.BlockSpec((NG, H), lambda i: (0, 0)),
                   pl.BlockSpec((1, NG), lambda i: (0, 0))],
        out_shape=[jax.ShapeDtypeStruct((NG, H), jnp.float32),
                   jax.ShapeDtypeStruct((1, NG), jnp.float32)],
    )(x64, m, gid2d, wo128, bo2d)


def kernel(x, edge_index, edge_x, tree_mess, tree_mess_tgt_nodes, graph_ids,
           W_i, W_h, W_o, b_o):
    n = x.shape[0]
    E = edge_index.shape[1]
    src = edge_index[0].astype(jnp.int32)
    dst = edge_index[1].astype(jnp.int32)
    depth_minus_1 = 3

    na_p = _sc_segment_sum(
        tree_mess, tree_mess_tgt_nodes.astype(jnp.int32).reshape(-1, 160), 160)
    node_alpha = na_p[:n]
    xp = x @ W_i[:, :x.shape[1]].T
    c = xp + node_alpha @ W_h.T

    nb = edge_x.shape[1]
    code = (edge_x @ jnp.float32(2.0) ** jnp.arange(nb)[:, None]
            ).astype(jnp.int32)
    bits = ((jnp.arange(32)[:, None] >> jnp.arange(nb)[None, :]) & 1
            ).astype(jnp.float32)
    lut = bits @ W_i[:, x.shape[1]:].T
    xc = jnp.concatenate([xp, c], axis=1)
    xcg = _sc_gather(xc, src.reshape(E // 320, 320), 2 * H, 320)
    msg, b = _edge_prep(xcg, code, lut)

    w2 = jnp.kron(jnp.eye(2, dtype=jnp.float32), W_h.T)
    dst2d = dst.reshape(E // 640, 640)
    dst_sc = dst.reshape(E // 160, 160)
    msg_p = msg.reshape(E // 2, 2 * H)
    b_p = b.reshape(E // 2, 2 * H)
    for _ in range(depth_minus_1):
        ns_p = _sc_segment_sum(msg_p.reshape(E, H), dst_sc, 160)
        g = _sc_gather(ns_p, dst2d, H, 640).reshape(E // 2, 2 * H)
        msg_p = _edge_iter(g, msg_p, b_p, w2)

    ns_p = _sc_segment_sum(msg_p.reshape(E, H), dst_sc, 160)
    m = (ns_p + na_p)[:n]

    x64 = jnp.pad(x, ((0, 0), (0, H - x.shape[1])))
    wo128 = jnp.concatenate(
        [jnp.pad(W_o[:, :x.shape[1]], ((0, 0), (0, H - x.shape[1]))),
         W_o[:, x.shape[1]:]], axis=1).T
    sums, cnts = _final(x64, m, graph_ids.reshape(n, 1).astype(jnp.int32),
                        wo128, b_o.reshape(1, H))
    return sums / jnp.maximum(cnts.reshape(NG, 1), 1.0)


# speedup vs baseline: 5.3759x; 1.3613x over previous
import functools

import jax
import jax.numpy as jnp
from jax import lax
from jax.experimental import pallas as pl
from jax.experimental.pallas import tpu as pltpu
from jax.experimental.pallas import tpu_sc as plsc

H = 64
NG = 256
NC, NS = 2, 16
NW = NC * NS
_PREC = jax.lax.Precision.HIGHEST


def _sc_gather(table, idx2d, width, ch):
    nch = idx2d.shape[0]
    e_out = nch * ch
    mesh = plsc.VectorSubcoreMesh(core_axis_name="c", subcore_axis_name="s")

    def body(table_hbm, idx_hbm, out_hbm, ib0, ib1, rb0, rb1,
             isem0, isem1, gsem0, gsem1, osem0, osem1):
        wid = lax.axis_index("s") * NC + lax.axis_index("c")
        ibufs, rbufs = (ib0, ib1), (rb0, rb1)
        isems, gsems, osems = (isem0, isem1), (gsem0, gsem1), (osem0, osem1)

        for b in (0, 1):
            ch0 = wid + NW * b

            @pl.when(ch0 < nch)
            def _():
                pltpu.make_async_copy(idx_hbm.at[ch0], ibufs[b], isems[b]).start()

        @pl.loop(wid, nch, step=2 * NW)
        def _(g):
            for b in (0, 1):
                cno = g + NW * b

                @pl.when(cno < nch)
                def _():
                    @pl.when(cno >= wid + 2 * NW)
                    def _():
                        pltpu.make_async_copy(
                            rbufs[b], out_hbm.at[pl.ds((cno - 2 * NW) * ch, ch)],
                            osems[b]).wait()

                    pltpu.make_async_copy(idx_hbm.at[cno], ibufs[b],
                                          isems[b]).wait()
                    pltpu.make_async_copy(table_hbm.at[ibufs[b]], rbufs[b],
                                          gsems[b]).start()
                    pltpu.make_async_copy(table_hbm.at[ibufs[b]], rbufs[b],
                                          gsems[b]).wait()

                    @pl.when(cno + 2 * NW < nch)
                    def _():
                        pltpu.make_async_copy(idx_hbm.at[cno + 2 * NW],
                                              ibufs[b], isems[b]).start()

                    pltpu.make_async_copy(
                        rbufs[b], out_hbm.at[pl.ds(cno * ch, ch)],
                        osems[b]).start()

        for b in (0, 1):
            last = ((nch - 1 - wid - NW * b) // (2 * NW)) * 2 * NW + wid + NW * b

            @pl.when((last >= 0) & (last < nch))
            def _():
                pltpu.make_async_copy(
                    rbufs[b], out_hbm.at[pl.ds(last * ch, ch)], osems[b]).wait()

    f = pl.kernel(
        body, mesh=mesh,
        out_type=jax.ShapeDtypeStruct((e_out, width), jnp.float32),
        compiler_params=pltpu.CompilerParams(use_tc_tiling_on_sc=False),
        scratch_types=[pltpu.VMEM((ch,), jnp.int32),
                       pltpu.VMEM((ch,), jnp.int32),
                       pltpu.VMEM((ch, width), jnp.float32),
                       pltpu.VMEM((ch, width), jnp.float32)]
        + [pltpu.SemaphoreType.DMA] * 6,
    )
    return f(table, idx2d)


NPAD = 50176
HW = H // 2
_SHARE = NPAD // NS


def _sc_segment_sum(data, idx2d, ch):
    nch = idx2d.shape[0]
    mesh = plsc.VectorSubcoreMesh(core_axis_name="c", subcore_axis_name="s")

    def body(data_hbm, idx_hbm, out_hbm, table, ib0, ib1, db0, db1,
             isem0, isem1, dsem0, dsem1):
        core = lax.axis_index("c")
        tid = lax.axis_index("s")
        ibufs, dbufs = (ib0, ib1), (db0, db1)
        isems, dsems = (isem0, isem1), (dsem0, dsem1)
        zvec = jnp.zeros((16,), jnp.float32)
        c0 = core * HW

        def start_loads(cno, b):
            pltpu.make_async_copy(idx_hbm.at[cno], ibufs[b], isems[b]).start()
            pltpu.make_async_copy(
                data_hbm.at[pl.ds(cno * ch, ch), pl.ds(c0, HW)], dbufs[b],
                dsems[b]).start()

        @pl.loop(0, ch)
        def _(r):
            for q in range(0, HW, 16):
                db0[r, pl.ds(q, 16)] = zvec

        nz = _SHARE // ch
        for z in range(nz):
            pltpu.sync_copy(db0, table.at[pl.ds(tid * _SHARE + z * ch, ch)])
        rem = _SHARE - nz * ch
        if rem:
            pltpu.sync_copy(db0.at[pl.ds(0, rem)],
                            table.at[pl.ds(tid * _SHARE + nz * ch, rem)])

        plsc.subcore_barrier()

        for b in (0, 1):
            cno = tid + NS * b

            @pl.when(cno < nch)
            def _():
                start_loads(cno, b)

        @pl.loop(tid, nch, step=2 * NS)
        def _(g):
            for b in (0, 1):
                cno = g + NS * b

                @pl.when(cno < nch)
                def _():
                    pltpu.make_async_copy(idx_hbm.at[cno], ibufs[b],
                                          isems[b]).wait()
                    pltpu.make_async_copy(
                        data_hbm.at[pl.ds(cno * ch, ch), pl.ds(c0, HW)],
                        dbufs[b], dsems[b]).wait()

                    pltpu.sync_copy(dbufs[b], table.at[ibufs[b]], add=True)

                    @pl.when(cno + 2 * NS < nch)
                    def _():
                        start_loads(cno + 2 * NS, b)

        plsc.subcore_barrier()

        for z in range((_SHARE + ch - 1) // ch):
            rows = min(ch, _SHARE - z * ch)
            start = tid * _SHARE + z * ch
            pltpu.sync_copy(table.at[pl.ds(start, rows)],
                            out_hbm.at[pl.ds(start, rows), pl.ds(c0, HW)])

    f = pl.kernel(
        body, mesh=mesh,
        out_type=jax.ShapeDtypeStruct((NPAD, H), jnp.float32),
        compiler_params=pltpu.CompilerParams(use_tc_tiling_on_sc=False),
        scratch_types=[pltpu.VMEM_SHARED((NPAD, HW), jnp.float32),
                       pltpu.VMEM((ch,), jnp.int32),
                       pltpu.VMEM((ch,), jnp.int32),
                       pltpu.VMEM((ch, HW), jnp.float32),
                       pltpu.VMEM((ch, HW), jnp.float32)]
        + [pltpu.SemaphoreType.DMA] * 4,
    )
    return f(data, idx2d)


def _prep_body(xcg_ref, code_ref, lut_ref, msg0_ref, b_ref):
    oneh = (code_ref[...] == jax.lax.broadcasted_iota(jnp.int32, (1, 32), 1)
            ).astype(jnp.float32)
    ew = jnp.dot(oneh, lut_ref[...], preferred_element_type=jnp.float32,
                 precision=_PREC)
    msg0_ref[...] = jnp.maximum(xcg_ref[:, :H] + ew, 0.0)
    b_ref[...] = xcg_ref[:, H:] + ew


def _edge_prep(xcg, code, lut):
    E = xcg.shape[0]
    BLK = 8000
    grid = (E // BLK,)
    bs = lambda w: pl.BlockSpec((BLK, w), lambda i: (i, 0))
    return pl.pallas_call(
        _prep_body,
        grid=grid,
        in_specs=[bs(2 * H), pl.BlockSpec((BLK, 1), lambda i: (i, 0)),
                  pl.BlockSpec((32, H), lambda i: (0, 0))],
        out_specs=[bs(H), bs(H)],
        out_shape=[jax.ShapeDtypeStruct((E, H), jnp.float32),
                   jax.ShapeDtypeStruct((E, H), jnp.float32)],
    )(xcg, code, lut)


def _iter_body(g_ref, m_ref, b_ref, w2_ref, o_ref):
    t = g_ref[...] - m_ref[...]
    tw = jnp.dot(t, w2_ref[...], preferred_element_type=jnp.float32,
                 precision=_PREC)
    accum = jnp.concatenate([tw[:, H:], tw[:, :H]], axis=1)
    o_ref[...] = jnp.maximum(b_ref[...] + accum, 0.0)


def _edge_iter(g, msg, b, w2):
    EP = g.shape[0]
    BLK = 8000
    grid = (EP // BLK,)
    bs = pl.BlockSpec((BLK, 2 * H), lambda i: (i, 0))
    return pl.pallas_call(
        _iter_body,
        grid=grid,
        in_specs=[bs, bs, bs, pl.BlockSpec((2 * H, 2 * H), lambda i: (0, 0))],
        out_specs=bs,
        out_shape=jax.ShapeDtypeStruct((EP, 2 * H), jnp.float32),
    )(g, msg, b, w2)


def _final_body(x64_ref, m_ref, gid_ref, wo_ref, bo_ref, sums_ref, cnts_ref):
    i = pl.program_id(0)

    @pl.when(i == 0)
    def _():
        sums_ref[...] = jnp.zeros_like(sums_ref)
        cnts_ref[...] = jnp.zeros_like(cnts_ref)

    xm = jnp.concatenate([x64_ref[...], m_ref[...]], axis=1)
    h = jnp.maximum(
        jnp.dot(xm, wo_ref[...], preferred_element_type=jnp.float32,
                precision=_PREC) + bo_ref[...], 0.0)
    oneh = (gid_ref[...] == jax.lax.broadcasted_iota(jnp.int32, (1, NG), 1)
            ).astype(jnp.float32)
    sums_ref[...] += jax.lax.dot_general(
        oneh, h, (((0,), (0,)), ((), ())), preferred_element_type=jnp.float32,
        precision=_PREC)
    cnts_ref[...] += jnp.sum(oneh, axis=0, keepdims=True)


def _final(x64, m, gid2d, wo128, bo2d):
    Nn = x64.shape[0]
    BLK = 2000
    grid = (Nn // BLK,)
    bs = lambda w: pl.BlockSpec((BLK, w), lambda i: (i, 0))
    return pl.pallas_call(
        _final_body,
        grid=grid,
        in_specs=[bs(H), bs(H), pl.BlockSpec((BLK, 1), lambda i: (i, 0)),
                  pl.BlockSpec((2 * H, H), lambda i: (0, 0)),
                  pl.BlockSpec((1, H), lambda i: (0, 0))],
        out_specs=[pl.BlockSpec((NG, H), lambda i: (0, 0)),
                   pl.BlockSpec((1, NG), lambda i: (0, 0))],
        out_shape=[jax.ShapeDtypeStruct((NG, H), jnp.float32),
                   jax.ShapeDtypeStruct((1, NG), jnp.float32)],
    )(x64, m, gid2d, wo128, bo2d)


def kernel(x, edge_index, edge_x, tree_mess, tree_mess_tgt_nodes, graph_ids,
           W_i, W_h, W_o, b_o):
    n = x.shape[0]
    E = edge_index.shape[1]
    src = edge_index[0].astype(jnp.int32)
    dst = edge_index[1].astype(jnp.int32)
    depth_minus_1 = 3

    na_p = _sc_segment_sum(
        tree_mess, tree_mess_tgt_nodes.astype(jnp.int32).reshape(-1, 160), 160)
    node_alpha = na_p[:n]
    xp = x @ W_i[:, :x.shape[1]].T
    c = xp + node_alpha @ W_h.T

    nb = edge_x.shape[1]
    code = (edge_x @ jnp.float32(2.0) ** jnp.arange(nb)[:, None]
            ).astype(jnp.int32)
    bits = ((jnp.arange(32)[:, None] >> jnp.arange(nb)[None, :]) & 1
            ).astype(jnp.float32)
    lut = bits @ W_i[:, x.shape[1]:].T
    xc = jnp.concatenate([xp, c], axis=1)
    xcg = _sc_gather(xc, src.reshape(E // 320, 320), 2 * H, 320)
    msg, b = _edge_prep(xcg, code, lut)

    w2 = jnp.kron(jnp.eye(2, dtype=jnp.float32), W_h.T)
    dst2d = dst.reshape(E // 640, 640)
    dst_sc = dst.reshape(E // 320, 320)
    msg_p = msg.reshape(E // 2, 2 * H)
    b_p = b.reshape(E // 2, 2 * H)
    for _ in range(depth_minus_1):
        ns_p = _sc_segment_sum(msg_p.reshape(E, H), dst_sc, 320)
        g = _sc_gather(ns_p, dst2d, H, 640).reshape(E // 2, 2 * H)
        msg_p = _edge_iter(g, msg_p, b_p, w2)

    ns_p = _sc_segment_sum(msg_p.reshape(E, H), dst_sc, 320)
    m = (ns_p + na_p)[:n]

    x64 = jnp.pad(x, ((0, 0), (0, H - x.shape[1])))
    wo128 = jnp.concatenate(
        [jnp.pad(W_o[:, :x.shape[1]], ((0, 0), (0, H - x.shape[1]))),
         W_o[:, x.shape[1]:]], axis=1).T
    sums, cnts = _final(x64, m, graph_ids.reshape(n, 1).astype(jnp.int32),
                        wo128, b_o.reshape(1, H))
    return sums / jnp.maximum(cnts.reshape(NG, 1), 1.0)


# speedup vs baseline: 5.3916x; 1.0029x over previous
import functools

import jax
import jax.numpy as jnp
from jax import lax
from jax.experimental import pallas as pl
from jax.experimental.pallas import tpu as pltpu
from jax.experimental.pallas import tpu_sc as plsc

H = 64
NG = 256
NC, NS = 2, 16
NW = NC * NS
_PREC = jax.lax.Precision.HIGHEST


def _sc_gather(table, idx2d, width, ch):
    nch = idx2d.shape[0]
    e_out = nch * ch
    mesh = plsc.VectorSubcoreMesh(core_axis_name="c", subcore_axis_name="s")

    def body(table_hbm, idx_hbm, out_hbm, ib0, ib1, rb0, rb1,
             isem0, isem1, gsem0, gsem1, osem0, osem1):
        wid = lax.axis_index("s") * NC + lax.axis_index("c")
        ibufs, rbufs = (ib0, ib1), (rb0, rb1)
        isems, gsems, osems = (isem0, isem1), (gsem0, gsem1), (osem0, osem1)

        for b in (0, 1):
            ch0 = wid + NW * b

            @pl.when(ch0 < nch)
            def _():
                pltpu.make_async_copy(idx_hbm.at[ch0], ibufs[b], isems[b]).start()

        @pl.loop(wid, nch, step=2 * NW)
        def _(g):
            for b in (0, 1):
                cno = g + NW * b

                @pl.when(cno < nch)
                def _():
                    @pl.when(cno >= wid + 2 * NW)
                    def _():
                        pltpu.make_async_copy(
                            rbufs[b], out_hbm.at[pl.ds((cno - 2 * NW) * ch, ch)],
                            osems[b]).wait()

                    pltpu.make_async_copy(idx_hbm.at[cno], ibufs[b],
                                          isems[b]).wait()
                    pltpu.make_async_copy(table_hbm.at[ibufs[b]], rbufs[b],
                                          gsems[b]).start()
                    pltpu.make_async_copy(table_hbm.at[ibufs[b]], rbufs[b],
                                          gsems[b]).wait()

                    @pl.when(cno + 2 * NW < nch)
                    def _():
                        pltpu.make_async_copy(idx_hbm.at[cno + 2 * NW],
                                              ibufs[b], isems[b]).start()

                    pltpu.make_async_copy(
                        rbufs[b], out_hbm.at[pl.ds(cno * ch, ch)],
                        osems[b]).start()

        for b in (0, 1):
            last = ((nch - 1 - wid - NW * b) // (2 * NW)) * 2 * NW + wid + NW * b

            @pl.when((last >= 0) & (last < nch))
            def _():
                pltpu.make_async_copy(
                    rbufs[b], out_hbm.at[pl.ds(last * ch, ch)], osems[b]).wait()

    f = pl.kernel(
        body, mesh=mesh,
        out_type=jax.ShapeDtypeStruct((e_out, width), jnp.float32),
        compiler_params=pltpu.CompilerParams(use_tc_tiling_on_sc=False),
        scratch_types=[pltpu.VMEM((ch,), jnp.int32),
                       pltpu.VMEM((ch,), jnp.int32),
                       pltpu.VMEM((ch, width), jnp.float32),
                       pltpu.VMEM((ch, width), jnp.float32)]
        + [pltpu.SemaphoreType.DMA] * 6,
    )
    return f(table, idx2d)


NPAD = 50176
HW = H // 2
_SHARE = NPAD // NS


def _sc_segment_sum(data, idx2d, ch):
    nch = idx2d.shape[0]
    mesh = plsc.VectorSubcoreMesh(core_axis_name="c", subcore_axis_name="s")

    def body(data_hbm, idx_hbm, out_hbm, table, ib0, ib1, db0, db1,
             isem0, isem1, dsem0, dsem1):
        core = lax.axis_index("c")
        tid = lax.axis_index("s")
        ibufs, dbufs = (ib0, ib1), (db0, db1)
        isems, dsems = (isem0, isem1), (dsem0, dsem1)
        zvec = jnp.zeros((16,), jnp.float32)
        c0 = core * HW

        def start_loads(cno, b):
            pltpu.make_async_copy(idx_hbm.at[cno], ibufs[b], isems[b]).start()
            pltpu.make_async_copy(
                data_hbm.at[pl.ds(cno * ch, ch), pl.ds(c0, HW)], dbufs[b],
                dsems[b]).start()

        @pl.loop(0, ch)
        def _(r):
            for q in range(0, HW, 16):
                db0[r, pl.ds(q, 16)] = zvec

        nz = _SHARE // ch
        for z in range(nz):
            pltpu.sync_copy(db0, table.at[pl.ds(tid * _SHARE + z * ch, ch)])
        rem = _SHARE - nz * ch
        if rem:
            pltpu.sync_copy(db0.at[pl.ds(0, rem)],
                            table.at[pl.ds(tid * _SHARE + nz * ch, rem)])

        plsc.subcore_barrier()

        for b in (0, 1):
            cno = tid + NS * b

            @pl.when(cno < nch)
            def _():
                start_loads(cno, b)

        @pl.loop(tid, nch, step=2 * NS)
        def _(g):
            for b in (0, 1):
                cno = g + NS * b

                @pl.when(cno < nch)
                def _():
                    pltpu.make_async_copy(idx_hbm.at[cno], ibufs[b],
                                          isems[b]).wait()
                    pltpu.make_async_copy(
                        data_hbm.at[pl.ds(cno * ch, ch), pl.ds(c0, HW)],
                        dbufs[b], dsems[b]).wait()

                    pltpu.sync_copy(dbufs[b], table.at[ibufs[b]], add=True)

                    @pl.when(cno + 2 * NS < nch)
                    def _():
                        start_loads(cno + 2 * NS, b)

        plsc.subcore_barrier()

        for z in range((_SHARE + ch - 1) // ch):
            rows = min(ch, _SHARE - z * ch)
            start = tid * _SHARE + z * ch
            pltpu.sync_copy(table.at[pl.ds(start, rows)],
                            out_hbm.at[pl.ds(start, rows), pl.ds(c0, HW)])

    f = pl.kernel(
        body, mesh=mesh,
        out_type=jax.ShapeDtypeStruct((NPAD, H), jnp.float32),
        compiler_params=pltpu.CompilerParams(use_tc_tiling_on_sc=False),
        scratch_types=[pltpu.VMEM_SHARED((NPAD, HW), jnp.float32),
                       pltpu.VMEM((ch,), jnp.int32),
                       pltpu.VMEM((ch,), jnp.int32),
                       pltpu.VMEM((ch, HW), jnp.float32),
                       pltpu.VMEM((ch, HW), jnp.float32)]
        + [pltpu.SemaphoreType.DMA] * 4,
    )
    return f(data, idx2d)


def _prep_body(xcg_ref, code_ref, lut_ref, msg0_ref, b_ref):
    oneh = (code_ref[...] == jax.lax.broadcasted_iota(jnp.int32, (1, 32), 1)
            ).astype(jnp.float32)
    ew = jnp.dot(oneh, lut_ref[...], preferred_element_type=jnp.float32,
                 precision=_PREC)
    v = xcg_ref[...]
    msg0_ref[...] = jnp.maximum(v[:, :H] + ew, 0.0)
    b_ref[...] = v[:, H:] + ew


def _edge_prep(xcg, code, lut):
    E = xcg.shape[0]
    BLK = 8000
    grid = (E // BLK,)
    bs = lambda w: pl.BlockSpec((BLK, w), lambda i: (i, 0))
    return pl.pallas_call(
        _prep_body,
        grid=grid,
        in_specs=[bs(2 * H), pl.BlockSpec((BLK, 1), lambda i: (i, 0)),
                  pl.BlockSpec((32, H), lambda i: (0, 0))],
        out_specs=[bs(H), bs(H)],
        out_shape=[jax.ShapeDtypeStruct((E, H), jnp.float32),
                   jax.ShapeDtypeStruct((E, H), jnp.float32)],
    )(xcg, code, lut)


def _iter_body(g_ref, m_ref, b_ref, wh_ref, o_ref):
    t = g_ref[...] - m_ref[...]
    tw = jnp.dot(t, wh_ref[...], preferred_element_type=jnp.float32,
                 precision=_PREC)
    o_ref[...] = jnp.maximum(b_ref[...] + tw, 0.0)


def _edge_iter(g, msg, b, wh):
    Ep = g.shape[0]
    BLK = 8000
    grid = (Ep // BLK,)
    bs = pl.BlockSpec((BLK, 2 * H), lambda i: (i, 0))
    return pl.pallas_call(
        _iter_body,
        grid=grid,
        in_specs=[bs, bs, bs, pl.BlockSpec((2 * H, 2 * H), lambda i: (0, 0))],
        out_specs=bs,
        out_shape=jax.ShapeDtypeStruct((Ep, 2 * H), jnp.float32),
    )(g, msg, b, wh)


def _final_body(x64_ref, m_ref, gid_ref, wo_ref, bo_ref, sums_ref, cnts_ref):
    i = pl.program_id(0)

    @pl.when(i == 0)
    def _():
        sums_ref[...] = jnp.zeros_like(sums_ref)
        cnts_ref[...] = jnp.zeros_like(cnts_ref)

    xm = jnp.concatenate([x64_ref[...], m_ref[...]], axis=1)
    h = jnp.maximum(
        jnp.dot(xm, wo_ref[...], preferred_element_type=jnp.float32,
                precision=_PREC) + bo_ref[...], 0.0)
    oneh = (gid_ref[...] == jax.lax.broadcasted_iota(jnp.int32, (1, NG), 1)
            ).astype(jnp.float32)
    sums_ref[...] += jax.lax.dot_general(
        oneh, h, (((0,), (0,)), ((), ())), preferred_element_type=jnp.float32,
        precision=_PREC)
    cnts_ref[...] += jnp.sum(oneh, axis=0, keepdims=True)


def _final(x64, m, gid2d, wo128, bo2d):
    Nn = x64.shape[0]
    BLK = 2000
    grid = (Nn // BLK,)
    bs = lambda w: pl.BlockSpec((BLK, w), lambda i: (i, 0))
    return pl.pallas_call(
        _final_body,
        grid=grid,
        in_specs=[bs(H), bs(H), pl.BlockSpec((BLK, 1), lambda i: (i, 0)),
                  pl.BlockSpec((2 * H, H), lambda i: (0, 0)),
                  pl.BlockSpec((1, H), lambda i: (0, 0))],
        out_specs=[pl.BlockSpec((NG, H), lambda i: (0, 0)),
                   pl.BlockSpec((1, NG), lambda i: (0, 0))],
        out_shape=[jax.ShapeDtypeStruct((NG, H), jnp.float32),
                   jax.ShapeDtypeStruct((1, NG), jnp.float32)],
    )(x64, m, gid2d, wo128, bo2d)


def kernel(x, edge_index, edge_x, tree_mess, tree_mess_tgt_nodes, graph_ids,
           W_i, W_h, W_o, b_o):
    n = x.shape[0]
    E = edge_index.shape[1]
    src = edge_index[0].astype(jnp.int32)
    dst = edge_index[1].astype(jnp.int32)
    depth_minus_1 = 3

    na_p = _sc_segment_sum(
        tree_mess, tree_mess_tgt_nodes.astype(jnp.int32).reshape(-1, 160), 160)
    node_alpha = na_p[:n]
    xp = x @ W_i[:, :x.shape[1]].T
    c = xp + node_alpha @ W_h.T

    nb = edge_x.shape[1]
    code = (edge_x @ jnp.float32(2.0) ** jnp.arange(nb)[:, None]
            ).astype(jnp.int32)
    bits = ((jnp.arange(32)[:, None] >> jnp.arange(nb)[None, :]) & 1
            ).astype(jnp.float32)
    lut = bits @ W_i[:, x.shape[1]:].T
    xc = jnp.concatenate([xp, c], axis=1)
    xcg = _sc_gather(xc, src.reshape(E // 320, 320), 2 * H, 320)
    msg, b = _edge_prep(xcg, code, lut)

    w2 = jnp.kron(jnp.float32([[0, 1], [1, 0]]), W_h.T)
    dst2d = dst.reshape(E // 640, 640)
    dst_sc = dst.reshape(E // 320, 320)
    msg_p = msg.reshape(E // 2, 2 * H)
    b_p = b.reshape(E // 2, 2 * H)
    for _ in range(depth_minus_1):
        ns_p = _sc_segment_sum(msg_p.reshape(E, H), dst_sc, 320)
        g = _sc_gather(ns_p, dst2d, H, 640).reshape(E // 2, 2 * H)
        msg_p = _edge_iter(g, msg_p, b_p, w2)

    ns_p = _sc_segment_sum(msg_p.reshape(E, H), dst_sc, 320)
    m = (ns_p + na_p)[:n]

    x64 = jnp.pad(x, ((0, 0), (0, H - x.shape[1])))
    wo128 = jnp.concatenate(
        [jnp.pad(W_o[:, :x.shape[1]], ((0, 0), (0, H - x.shape[1]))),
         W_o[:, x.shape[1]:]], axis=1).T
    sums, cnts = _final(x64, m, graph_ids.reshape(n, 1).astype(jnp.int32),
                        wo128, b_o.reshape(1, H))
    return sums / jnp.maximum(cnts.reshape(NG, 1), 1.0)


# speedup vs baseline: 5.8343x; 1.0821x over previous
import functools

import jax
import jax.numpy as jnp
from jax import lax
from jax.experimental import pallas as pl
from jax.experimental.pallas import tpu as pltpu
from jax.experimental.pallas import tpu_sc as plsc

H = 64
NG = 256
NC, NS = 2, 16
NW = NC * NS
_PREC = jax.lax.Precision.HIGHEST


def _sc_gather(table, idx2d, width, ch):
    nch = idx2d.shape[0]
    e_out = nch * ch
    mesh = plsc.VectorSubcoreMesh(core_axis_name="c", subcore_axis_name="s")

    def body(table_hbm, idx_hbm, out_hbm, ib0, ib1, rb0, rb1,
             isem0, isem1, gsem0, gsem1, osem0, osem1):
        wid = lax.axis_index("s") * NC + lax.axis_index("c")
        ibufs, rbufs = (ib0, ib1), (rb0, rb1)
        isems, gsems, osems = (isem0, isem1), (gsem0, gsem1), (osem0, osem1)

        for b in (0, 1):
            ch0 = wid + NW * b

            @pl.when(ch0 < nch)
            def _():
                pltpu.make_async_copy(idx_hbm.at[ch0], ibufs[b], isems[b]).start()

        @pl.loop(wid, nch, step=2 * NW)
        def _(g):
            for b in (0, 1):
                cno = g + NW * b

                @pl.when(cno < nch)
                def _():
                    @pl.when(cno >= wid + 2 * NW)
                    def _():
                        pltpu.make_async_copy(
                            rbufs[b], out_hbm.at[pl.ds((cno - 2 * NW) * ch, ch)],
                            osems[b]).wait()

                    pltpu.make_async_copy(idx_hbm.at[cno], ibufs[b],
                                          isems[b]).wait()
                    pltpu.make_async_copy(table_hbm.at[ibufs[b]], rbufs[b],
                                          gsems[b]).start()
                    pltpu.make_async_copy(table_hbm.at[ibufs[b]], rbufs[b],
                                          gsems[b]).wait()

                    @pl.when(cno + 2 * NW < nch)
                    def _():
                        pltpu.make_async_copy(idx_hbm.at[cno + 2 * NW],
                                              ibufs[b], isems[b]).start()

                    pltpu.make_async_copy(
                        rbufs[b], out_hbm.at[pl.ds(cno * ch, ch)],
                        osems[b]).start()

        for b in (0, 1):
            last = ((nch - 1 - wid - NW * b) // (2 * NW)) * 2 * NW + wid + NW * b

            @pl.when((last >= 0) & (last < nch))
            def _():
                pltpu.make_async_copy(
                    rbufs[b], out_hbm.at[pl.ds(last * ch, ch)], osems[b]).wait()

    f = pl.kernel(
        body, mesh=mesh,
        out_type=jax.ShapeDtypeStruct((e_out, width), jnp.float32),
        compiler_params=pltpu.CompilerParams(use_tc_tiling_on_sc=False),
        scratch_types=[pltpu.VMEM((ch,), jnp.int32),
                       pltpu.VMEM((ch,), jnp.int32),
                       pltpu.VMEM((ch, width), jnp.float32),
                       pltpu.VMEM((ch, width), jnp.float32)]
        + [pltpu.SemaphoreType.DMA] * 6,
    )
    return f(table, idx2d)


NPAD = 50176
HW = H // 2
_SHARE = NPAD // NS


def _sc_segment_sum(data, idx2d, ch):
    nch = idx2d.shape[0]
    mesh = plsc.VectorSubcoreMesh(core_axis_name="c", subcore_axis_name="s")

    def body(data_hbm, idx_hbm, out_hbm, table, ib0, ib1, db0, db1,
             isem0, isem1, dsem0, dsem1):
        core = lax.axis_index("c")
        tid = lax.axis_index("s")
        ibufs, dbufs = (ib0, ib1), (db0, db1)
        isems, dsems = (isem0, isem1), (dsem0, dsem1)
        zvec = jnp.zeros((16,), jnp.float32)
        c0 = core * HW

        def start_loads(cno, b):
            pltpu.make_async_copy(idx_hbm.at[cno], ibufs[b], isems[b]).start()
            pltpu.make_async_copy(
                data_hbm.at[pl.ds(cno * ch, ch), pl.ds(c0, HW)], dbufs[b],
                dsems[b]).start()

        @pl.loop(0, ch)
        def _(r):
            for q in range(0, HW, 16):
                db0[r, pl.ds(q, 16)] = zvec

        nz = _SHARE // ch
        for z in range(nz):
            pltpu.sync_copy(db0, table.at[pl.ds(tid * _SHARE + z * ch, ch)])
        rem = _SHARE - nz * ch
        if rem:
            pltpu.sync_copy(db0.at[pl.ds(0, rem)],
                            table.at[pl.ds(tid * _SHARE + nz * ch, rem)])

        plsc.subcore_barrier()

        for b in (0, 1):
            cno = tid + NS * b

            @pl.when(cno < nch)
            def _():
                start_loads(cno, b)

        @pl.loop(tid, nch, step=2 * NS)
        def _(g):
            for b in (0, 1):
                cno = g + NS * b

                @pl.when(cno < nch)
                def _():
                    pltpu.make_async_copy(idx_hbm.at[cno], ibufs[b],
                                          isems[b]).wait()
                    pltpu.make_async_copy(
                        data_hbm.at[pl.ds(cno * ch, ch), pl.ds(c0, HW)],
                        dbufs[b], dsems[b]).wait()

                    pltpu.sync_copy(dbufs[b], table.at[ibufs[b]], add=True)

                    @pl.when(cno + 2 * NS < nch)
                    def _():
                        start_loads(cno + 2 * NS, b)

        plsc.subcore_barrier()

        for z in range((_SHARE + ch - 1) // ch):
            rows = min(ch, _SHARE - z * ch)
            start = tid * _SHARE + z * ch
            pltpu.sync_copy(table.at[pl.ds(start, rows)],
                            out_hbm.at[pl.ds(start, rows), pl.ds(c0, HW)])

    f = pl.kernel(
        body, mesh=mesh,
        out_type=jax.ShapeDtypeStruct((NPAD, H), jnp.float32),
        compiler_params=pltpu.CompilerParams(use_tc_tiling_on_sc=False),
        scratch_types=[pltpu.VMEM_SHARED((NPAD, HW), jnp.float32),
                       pltpu.VMEM((ch,), jnp.int32),
                       pltpu.VMEM((ch,), jnp.int32),
                       pltpu.VMEM((ch, HW), jnp.float32),
                       pltpu.VMEM((ch, HW), jnp.float32)]
        + [pltpu.SemaphoreType.DMA] * 4,
    )
    return f(data, idx2d)


def _sc_segsum_gather(data, idx2d, ch):
    nch = idx2d.shape[0]
    e_out = nch * ch
    mesh = plsc.VectorSubcoreMesh(core_axis_name="c", subcore_axis_name="s")

    def body(data_hbm, idx_hbm, out_hbm, table, ib0, ib1, db0, db1,
             isem0, isem1, dsem0, dsem1, gsem0, gsem1):
        core = lax.axis_index("c")
        tid = lax.axis_index("s")
        ibufs, dbufs = (ib0, ib1), (db0, db1)
        isems, dsems, gsems = (isem0, isem1), (dsem0, dsem1), (gsem0, gsem1)
        zvec = jnp.zeros((16,), jnp.float32)
        c0 = core * HW

        def start_loads(cno, b):
            pltpu.make_async_copy(idx_hbm.at[cno], ibufs[b], isems[b]).start()
            pltpu.make_async_copy(
                data_hbm.at[pl.ds(cno * ch, ch), pl.ds(c0, HW)], dbufs[b],
                dsems[b]).start()

        @pl.loop(0, ch)
        def _(r):
            for q in range(0, HW, 16):
                db0[r, pl.ds(q, 16)] = zvec

        nz = _SHARE // ch
        for z in range(nz):
            pltpu.sync_copy(db0, table.at[pl.ds(tid * _SHARE + z * ch, ch)])
        rem = _SHARE - nz * ch
        if rem:
            pltpu.sync_copy(db0.at[pl.ds(0, rem)],
                            table.at[pl.ds(tid * _SHARE + nz * ch, rem)])

        plsc.subcore_barrier()

        for b in (0, 1):
            cno = tid + NS * b

            @pl.when(cno < nch)
            def _():
                start_loads(cno, b)

        @pl.loop(tid, nch, step=2 * NS)
        def _(g):
            for b in (0, 1):
                cno = g + NS * b

                @pl.when(cno < nch)
                def _():
                    pltpu.make_async_copy(idx_hbm.at[cno], ibufs[b],
                                          isems[b]).wait()
                    pltpu.make_async_copy(
                        data_hbm.at[pl.ds(cno * ch, ch), pl.ds(c0, HW)],
                        dbufs[b], dsems[b]).wait()

                    pltpu.sync_copy(dbufs[b], table.at[ibufs[b]], add=True)

                    @pl.when(cno + 2 * NS < nch)
                    def _():
                        start_loads(cno + 2 * NS, b)

        plsc.subcore_barrier()

        for b in (0, 1):
            cno = tid + NS * b

            @pl.when(cno < nch)
            def _():
                pltpu.make_async_copy(idx_hbm.at[cno], ibufs[b],
                                      isems[b]).start()

        @pl.loop(tid, nch, step=2 * NS)
        def _(g):
            for b in (0, 1):
                cno = g + NS * b

                @pl.when(cno < nch)
                def _():
                    @pl.when(cno >= tid + 2 * NS)
                    def _():
                        pltpu.make_async_copy(
                            dbufs[b],
                            out_hbm.at[pl.ds((cno - 2 * NS) * ch, ch),
                                       pl.ds(c0, HW)],
                            dsems[b]).wait()

                    pltpu.make_async_copy(idx_hbm.at[cno], ibufs[b],
                                          isems[b]).wait()
                    pltpu.make_async_copy(table.at[ibufs[b]], dbufs[b],
                                          gsems[b]).start()
                    pltpu.make_async_copy(table.at[ibufs[b]], dbufs[b],
                                          gsems[b]).wait()

                    @pl.when(cno + 2 * NS < nch)
                    def _():
                        pltpu.make_async_copy(idx_hbm.at[cno + 2 * NS],
                                              ibufs[b], isems[b]).start()

                    pltpu.make_async_copy(
                        dbufs[b],
                        out_hbm.at[pl.ds(cno * ch, ch), pl.ds(c0, HW)],
                        dsems[b]).start()

        for b in (0, 1):
            last = ((nch - 1 - tid - NS * b) // (2 * NS)) * 2 * NS + tid + NS * b

            @pl.when((last >= 0) & (last < nch))
            def _():
                pltpu.make_async_copy(
                    dbufs[b],
                    out_hbm.at[pl.ds(last * ch, ch), pl.ds(c0, HW)],
                    dsems[b]).wait()

    f = pl.kernel(
        body, mesh=mesh,
        out_type=jax.ShapeDtypeStruct((e_out, H), jnp.float32),
        compiler_params=pltpu.CompilerParams(use_tc_tiling_on_sc=False),
        scratch_types=[pltpu.VMEM_SHARED((NPAD, HW), jnp.float32),
                       pltpu.VMEM((ch,), jnp.int32),
                       pltpu.VMEM((ch,), jnp.int32),
                       pltpu.VMEM((ch, HW), jnp.float32),
                       pltpu.VMEM((ch, HW), jnp.float32)]
        + [pltpu.SemaphoreType.DMA] * 6,
    )
    return f(data, idx2d)


def _prep_body(xcg_ref, code_ref, lut_ref, msg0_ref, b_ref):
    oneh = (code_ref[...] == jax.lax.broadcasted_iota(jnp.int32, (1, 32), 1)
            ).astype(jnp.float32)
    ew = jnp.dot(oneh, lut_ref[...], preferred_element_type=jnp.float32,
                 precision=_PREC)
    v = xcg_ref[...]
    msg0_ref[...] = jnp.maximum(v[:, :H] + ew, 0.0)
    b_ref[...] = v[:, H:] + ew


def _edge_prep(xcg, code, lut):
    E = xcg.shape[0]
    BLK = 8000
    grid = (E // BLK,)
    bs = lambda w: pl.BlockSpec((BLK, w), lambda i: (i, 0))
    return pl.pallas_call(
        _prep_body,
        grid=grid,
        in_specs=[bs(2 * H), pl.BlockSpec((BLK, 1), lambda i: (i, 0)),
                  pl.BlockSpec((32, H), lambda i: (0, 0))],
        out_specs=[bs(H), bs(H)],
        out_shape=[jax.ShapeDtypeStruct((E, H), jnp.float32),
                   jax.ShapeDtypeStruct((E, H), jnp.float32)],
    )(xcg, code, lut)


def _iter_body(g_ref, m_ref, b_ref, wh_ref, o_ref):
    t = g_ref[...] - m_ref[...]
    tw = jnp.dot(t, wh_ref[...], preferred_element_type=jnp.float32,
                 precision=_PREC)
    o_ref[...] = jnp.maximum(b_ref[...] + tw, 0.0)


def _edge_iter(g, msg, b, wh):
    Ep = g.shape[0]
    BLK = 8000
    grid = (Ep // BLK,)
    bs = pl.BlockSpec((BLK, 2 * H), lambda i: (i, 0))
    return pl.pallas_call(
        _iter_body,
        grid=grid,
        in_specs=[bs, bs, bs, pl.BlockSpec((2 * H, 2 * H), lambda i: (0, 0))],
        out_specs=bs,
        out_shape=jax.ShapeDtypeStruct((Ep, 2 * H), jnp.float32),
    )(g, msg, b, wh)


def _final_body(x64_ref, m_ref, gid_ref, wo_ref, bo_ref, sums_ref, cnts_ref):
    i = pl.program_id(0)

    @pl.when(i == 0)
    def _():
        sums_ref[...] = jnp.zeros_like(sums_ref)
        cnts_ref[...] = jnp.zeros_like(cnts_ref)

    xm = jnp.concatenate([x64_ref[...], m_ref[...]], axis=1)
    h = jnp.maximum(
        jnp.dot(xm, wo_ref[...], preferred_element_type=jnp.float32,
                precision=_PREC) + bo_ref[...], 0.0)
    oneh = (gid_ref[...] == jax.lax.broadcasted_iota(jnp.int32, (1, NG), 1)
            ).astype(jnp.float32)
    sums_ref[...] += jax.lax.dot_general(
        oneh, h, (((0,), (0,)), ((), ())), preferred_element_type=jnp.float32,
        precision=_PREC)
    cnts_ref[...] += jnp.sum(oneh, axis=0, keepdims=True)


def _final(x64, m, gid2d, wo128, bo2d):
    Nn = x64.shape[0]
    BLK = 2000
    grid = (Nn // BLK,)
    bs = lambda w: pl.BlockSpec((BLK, w), lambda i: (i, 0))
    return pl.pallas_call(
        _final_body,
        grid=grid,
        in_specs=[bs(H), bs(H), pl.BlockSpec((BLK, 1), lambda i: (i, 0)),
                  pl.BlockSpec((2 * H, H), lambda i: (0, 0)),
                  pl.BlockSpec((1, H), lambda i: (0, 0))],
        out_specs=[pl.BlockSpec((NG, H), lambda i: (0, 0)),
                   pl.BlockSpec((1, NG), lambda i: (0, 0))],
        out_shape=[jax.ShapeDtypeStruct((NG, H), jnp.float32),
                   jax.ShapeDtypeStruct((1, NG), jnp.float32)],
    )(x64, m, gid2d, wo128, bo2d)


def kernel(x, edge_index, edge_x, tree_mess, tree_mess_tgt_nodes, graph_ids,
           W_i, W_h, W_o, b_o):
    n = x.shape[0]
    E = edge_index.shape[1]
    src = edge_index[0].astype(jnp.int32)
    dst = edge_index[1].astype(jnp.int32)
    depth_minus_1 = 3

    na_p = _sc_segment_sum(
        tree_mess, tree_mess_tgt_nodes.astype(jnp.int32).reshape(-1, 160), 160)
    node_alpha = na_p[:n]
    xp = x @ W_i[:, :x.shape[1]].T
    c = xp + node_alpha @ W_h.T

    nb = edge_x.shape[1]
    code = (edge_x @ jnp.float32(2.0) ** jnp.arange(nb)[:, None]
            ).astype(jnp.int32)
    bits = ((jnp.arange(32)[:, None] >> jnp.arange(nb)[None, :]) & 1
            ).astype(jnp.float32)
    lut = bits @ W_i[:, x.shape[1]:].T
    xc = jnp.concatenate([xp, c], axis=1)
    xcg = _sc_gather(xc, src.reshape(E // 320, 320), 2 * H, 320)
    msg, b = _edge_prep(xcg, code, lut)

    w2 = jnp.kron(jnp.float32([[0, 1], [1, 0]]), W_h.T)
    dst_sc = dst.reshape(E // 320, 320)
    msg_p = msg.reshape(E // 2, 2 * H)
    b_p = b.reshape(E // 2, 2 * H)
    for _ in range(depth_minus_1):
        g = _sc_segsum_gather(msg_p.reshape(E, H), dst_sc, 320
                              ).reshape(E // 2, 2 * H)
        msg_p = _edge_iter(g, msg_p, b_p, w2)

    ns_p = _sc_segment_sum(msg_p.reshape(E, H), dst_sc, 320)
    m = (ns_p + na_p)[:n]

    x64 = jnp.pad(x, ((0, 0), (0, H - x.shape[1])))
    wo128 = jnp.concatenate(
        [jnp.pad(W_o[:, :x.shape[1]], ((0, 0), (0, H - x.shape[1]))),
         W_o[:, x.shape[1]:]], axis=1).T
    sums, cnts = _final(x64, m, graph_ids.reshape(n, 1).astype(jnp.int32),
                        wo128, b_o.reshape(1, H))
    return sums / jnp.maximum(cnts.reshape(NG, 1), 1.0)
